# baseline (speedup 1.0000x reference)
import functools

import jax
import jax.numpy as jnp
from jax import lax
from jax.experimental import pallas as pl
from jax.experimental.pallas import tpu as pltpu
from jax.experimental.pallas import tpu_sc as plsc

N_OBJS = 1000000
D_OBJ = 64
N_ART = 10
D_ART = 32
BATCH = 16384

NC = 2
NS = 16
NW = NC * NS
B_PER_W = BATCH // NW
CHUNK = 128
N_CHUNKS = B_PER_W // CHUNK

TBLK = 8192

_mesh = plsc.VectorSubcoreMesh(core_axis_name="c", subcore_axis_name="s")


def _transpose_fuse_kernel(ws_ref, wa_ref, o_ref):
    o_ref[:, 0:D_OBJ] = jnp.transpose(ws_ref[...])
    o_ref[:, D_OBJ:2 * D_OBJ] = jnp.transpose(wa_ref[...])


def _art_kernel(ai_ref, wr_ref, o_ref):
    ids = ai_ref[...].reshape(BATCH, 1)
    onehot = (ids == lax.broadcasted_iota(jnp.int32, (BATCH, N_ART), 1)
              ).astype(jnp.float32)
    o_ref[...] = jnp.dot(onehot, wr_ref[...],
                         preferred_element_type=jnp.float32,
                         precision=lax.Precision.HIGHEST)


@jax.jit
def _lookup(instance_id, articulation_id, W_shape, W_app, W_art):
    wst = W_shape.T
    wat = W_app.T

    n_tblk = pl.cdiv(N_OBJS, TBLK)
    fused = pl.pallas_call(
        _transpose_fuse_kernel,
        grid=(n_tblk,),
        in_specs=[
            pl.BlockSpec((D_OBJ, TBLK), lambda i: (0, i)),
            pl.BlockSpec((D_OBJ, TBLK), lambda i: (0, i)),
        ],
        out_specs=pl.BlockSpec((TBLK, 2 * D_OBJ), lambda i: (i, 0)),
        out_shape=jax.ShapeDtypeStruct((N_OBJS, 2 * D_OBJ), jnp.float32),
        compiler_params=pltpu.CompilerParams(
            dimension_semantics=("parallel",)),
    )(wst, wat)

    articulation = pl.pallas_call(
        _art_kernel,
        in_specs=[
            pl.BlockSpec((1, BATCH), lambda: (0, 0)),
            pl.BlockSpec((N_ART, D_ART), lambda: (0, 0)),
        ],
        out_specs=pl.BlockSpec((BATCH, D_ART), lambda: (0, 0)),
        out_shape=jax.ShapeDtypeStruct((BATCH, D_ART), jnp.float32),
    )(articulation_id.reshape(1, BATCH), W_art)

    @functools.partial(
        pl.kernel,
        out_type=jax.ShapeDtypeStruct((BATCH, 2 * D_OBJ), jnp.float32),
        mesh=_mesh,
        scratch_types=[
            pltpu.VMEM((B_PER_W,), jnp.int32),
            pltpu.VMEM((B_PER_W, 2 * D_OBJ), jnp.float32),
            pltpu.SemaphoreType.DMA,
            pltpu.SemaphoreType.DMA,
        ],
    )
    def _gather(tab_hbm, ii_hbm, out_hbm, ii_v, rows_v, sem_g, sem_o):
        wid = lax.axis_index("s") * NC + lax.axis_index("c")
        base = wid * B_PER_W

        pltpu.sync_copy(ii_hbm.at[pl.ds(base, B_PER_W)], ii_v)

        for t in range(N_CHUNKS):
            cb = t * CHUNK
            pltpu.async_copy(
                tab_hbm.at[ii_v.at[pl.ds(cb, CHUNK)]],
                rows_v.at[pl.ds(cb, CHUNK)], sem_g)
        for t in range(N_CHUNKS):
            cb = t * CHUNK
            pltpu.make_async_copy(
                tab_hbm.at[ii_v.at[pl.ds(cb, CHUNK)]],
                rows_v.at[pl.ds(cb, CHUNK)], sem_g).wait()

        pltpu.sync_copy(rows_v, out_hbm.at[pl.ds(base, B_PER_W)])

    fused_out = _gather(fused, instance_id)
    density = fused_out[:, 0:D_OBJ]
    color = fused_out[:, D_OBJ:2 * D_OBJ]
    return (density, color, articulation)


def kernel(instance_id, articulation_id, W_shape, W_app, W_art):
    return _lookup(
        instance_id.astype(jnp.int32),
        articulation_id.astype(jnp.int32),
        W_shape,
        W_app,
        W_art,
    )


# speedup vs baseline: 1.0624x; 1.0624x over previous
import functools

import jax
import jax.numpy as jnp
from jax import lax
from jax.experimental import pallas as pl
from jax.experimental.pallas import tpu as pltpu
from jax.experimental.pallas import tpu_sc as plsc

N_OBJS = 1000000
D_OBJ = 64
N_ART = 10
D_ART = 32
BATCH = 16384

NC = 2
NS = 16
NW = NC * NS
B_PER_W = BATCH // NW
CHUNK = 128
N_CHUNKS = B_PER_W // CHUNK

TBLK = 16384

_mesh = plsc.VectorSubcoreMesh(core_axis_name="c", subcore_axis_name="s")


def _transpose_fuse_kernel(ws_ref, wa_ref, o_ref):
    eye = (lax.broadcasted_iota(jnp.int32, (D_OBJ, D_OBJ), 0)
           == lax.broadcasted_iota(jnp.int32, (D_OBJ, D_OBJ), 1)).astype(jnp.float32)
    dn = (((0,), (0,)), ((), ()))
    o_ref[:, 0:D_OBJ] = lax.dot_general(
        ws_ref[...], eye, dn, preferred_element_type=jnp.float32)
    o_ref[:, D_OBJ:2 * D_OBJ] = lax.dot_general(
        wa_ref[...], eye, dn, preferred_element_type=jnp.float32)


def _art_kernel(ai_ref, wr_ref, o_ref):
    ids = ai_ref[...].reshape(BATCH, 1)
    onehot = (ids == lax.broadcasted_iota(jnp.int32, (BATCH, N_ART), 1)
              ).astype(jnp.float32)
    o_ref[...] = jnp.dot(onehot, wr_ref[...],
                         preferred_element_type=jnp.float32,
                         precision=lax.Precision.HIGHEST)


@jax.jit
def _lookup(instance_id, articulation_id, W_shape, W_app, W_art):
    wst = W_shape.T
    wat = W_app.T

    n_tblk = pl.cdiv(N_OBJS, TBLK)
    fused = pl.pallas_call(
        _transpose_fuse_kernel,
        grid=(n_tblk,),
        in_specs=[
            pl.BlockSpec((D_OBJ, TBLK), lambda i: (0, i)),
            pl.BlockSpec((D_OBJ, TBLK), lambda i: (0, i)),
        ],
        out_specs=pl.BlockSpec((TBLK, 2 * D_OBJ), lambda i: (i, 0)),
        out_shape=jax.ShapeDtypeStruct((N_OBJS, 2 * D_OBJ), jnp.float32),
        compiler_params=pltpu.CompilerParams(
            dimension_semantics=("parallel",)),
    )(wst, wat)

    articulation = pl.pallas_call(
        _art_kernel,
        in_specs=[
            pl.BlockSpec((1, BATCH), lambda: (0, 0)),
            pl.BlockSpec((N_ART, D_ART), lambda: (0, 0)),
        ],
        out_specs=pl.BlockSpec((BATCH, D_ART), lambda: (0, 0)),
        out_shape=jax.ShapeDtypeStruct((BATCH, D_ART), jnp.float32),
    )(articulation_id.reshape(1, BATCH), W_art)

    @functools.partial(
        pl.kernel,
        out_type=jax.ShapeDtypeStruct((BATCH, 2 * D_OBJ), jnp.float32),
        mesh=_mesh,
        scratch_types=[
            pltpu.VMEM((B_PER_W,), jnp.int32),
            pltpu.VMEM((B_PER_W, 2 * D_OBJ), jnp.float32),
            pltpu.SemaphoreType.DMA,
            pltpu.SemaphoreType.DMA,
        ],
    )
    def _gather(tab_hbm, ii_hbm, out_hbm, ii_v, rows_v, sem_g, sem_o):
        wid = lax.axis_index("s") * NC + lax.axis_index("c")
        base = wid * B_PER_W

        pltpu.sync_copy(ii_hbm.at[pl.ds(base, B_PER_W)], ii_v)

        for t in range(N_CHUNKS):
            cb = t * CHUNK
            pltpu.async_copy(
                tab_hbm.at[ii_v.at[pl.ds(cb, CHUNK)]],
                rows_v.at[pl.ds(cb, CHUNK)], sem_g)
        for t in range(N_CHUNKS):
            cb = t * CHUNK
            pltpu.make_async_copy(
                tab_hbm.at[ii_v.at[pl.ds(cb, CHUNK)]],
                rows_v.at[pl.ds(cb, CHUNK)], sem_g).wait()

        pltpu.sync_copy(rows_v, out_hbm.at[pl.ds(base, B_PER_W)])

    fused_out = _gather(fused, instance_id)
    density = fused_out[:, 0:D_OBJ]
    color = fused_out[:, D_OBJ:2 * D_OBJ]
    return (density, color, articulation)


def kernel(instance_id, articulation_id, W_shape, W_app, W_art):
    return _lookup(
        instance_id.astype(jnp.int32),
        articulation_id.astype(jnp.int32),
        W_shape,
        W_app,
        W_art,
    )


# speedup vs baseline: 1.0634x; 1.0009x over previous
import functools

import jax
import jax.numpy as jnp
from jax import lax
from jax.experimental import pallas as pl
from jax.experimental.pallas import tpu as pltpu
from jax.experimental.pallas import tpu_sc as plsc

N_OBJS = 1000000
D_OBJ = 64
N_ART = 10
D_ART = 32
BATCH = 16384

NC = 2
NS = 16
NW = NC * NS
B_PER_W = BATCH // NW
CHUNK = 128
N_CHUNKS = B_PER_W // CHUNK

TBLK = 16384

_mesh = plsc.VectorSubcoreMesh(core_axis_name="c", subcore_axis_name="s")


def _transpose_fuse_kernel(ws_ref, wa_ref, o_ref):
    o_ref[:, 0:D_OBJ] = jnp.transpose(ws_ref[...])
    o_ref[:, D_OBJ:2 * D_OBJ] = jnp.transpose(wa_ref[...])


def _art_kernel(ai_ref, wr_ref, o_ref):
    ids = ai_ref[...].reshape(BATCH, 1)
    onehot = (ids == lax.broadcasted_iota(jnp.int32, (BATCH, N_ART), 1)
              ).astype(jnp.float32)
    o_ref[...] = jnp.dot(onehot, wr_ref[...],
                         preferred_element_type=jnp.float32,
                         precision=lax.Precision.HIGHEST)


@jax.jit
def _lookup(instance_id, articulation_id, W_shape, W_app, W_art):
    wst = W_shape.T
    wat = W_app.T

    n_tblk = pl.cdiv(N_OBJS, TBLK)
    fused = pl.pallas_call(
        _transpose_fuse_kernel,
        grid=(n_tblk,),
        in_specs=[
            pl.BlockSpec((D_OBJ, TBLK), lambda i: (0, i)),
            pl.BlockSpec((D_OBJ, TBLK), lambda i: (0, i)),
        ],
        out_specs=pl.BlockSpec((TBLK, 2 * D_OBJ), lambda i: (i, 0)),
        out_shape=jax.ShapeDtypeStruct((N_OBJS, 2 * D_OBJ), jnp.float32),
        compiler_params=pltpu.CompilerParams(
            dimension_semantics=("parallel",)),
    )(wst, wat)

    articulation = pl.pallas_call(
        _art_kernel,
        in_specs=[
            pl.BlockSpec((1, BATCH), lambda: (0, 0)),
            pl.BlockSpec((N_ART, D_ART), lambda: (0, 0)),
        ],
        out_specs=pl.BlockSpec((BATCH, D_ART), lambda: (0, 0)),
        out_shape=jax.ShapeDtypeStruct((BATCH, D_ART), jnp.float32),
    )(articulation_id.reshape(1, BATCH), W_art)

    @functools.partial(
        pl.kernel,
        out_type=jax.ShapeDtypeStruct((BATCH, 2 * D_OBJ), jnp.float32),
        mesh=_mesh,
        scratch_types=[
            pltpu.VMEM((B_PER_W,), jnp.int32),
            pltpu.VMEM((B_PER_W, 2 * D_OBJ), jnp.float32),
            pltpu.SemaphoreType.DMA,
            pltpu.SemaphoreType.DMA,
        ],
    )
    def _gather(tab_hbm, ii_hbm, out_hbm, ii_v, rows_v, sem_g, sem_o):
        wid = lax.axis_index("s") * NC + lax.axis_index("c")
        base = wid * B_PER_W

        pltpu.sync_copy(ii_hbm.at[pl.ds(base, B_PER_W)], ii_v)

        for t in range(N_CHUNKS):
            cb = t * CHUNK
            pltpu.async_copy(
                tab_hbm.at[ii_v.at[pl.ds(cb, CHUNK)]],
                rows_v.at[pl.ds(cb, CHUNK)], sem_g)
        for t in range(N_CHUNKS):
            cb = t * CHUNK
            pltpu.make_async_copy(
                tab_hbm.at[ii_v.at[pl.ds(cb, CHUNK)]],
                rows_v.at[pl.ds(cb, CHUNK)], sem_g).wait()

        pltpu.sync_copy(rows_v, out_hbm.at[pl.ds(base, B_PER_W)])

    fused_out = _gather(fused, instance_id)
    density = fused_out[:, 0:D_OBJ]
    color = fused_out[:, D_OBJ:2 * D_OBJ]
    return (density, color, articulation)


def kernel(instance_id, articulation_id, W_shape, W_app, W_art):
    return _lookup(
        instance_id.astype(jnp.int32),
        articulation_id.astype(jnp.int32),
        W_shape,
        W_app,
        W_art,
    )
